# baseline (speedup 1.0000x reference)
import functools

import jax
import jax.numpy as jnp
from jax.experimental import pallas as pl
from jax.experimental.pallas import tpu as pltpu

B = 64
D = 8732
C = 21
NOBJ = 8
THRESHOLD = 0.5
NEG_POS = 3
ALPHA = 1.0

G = 8
CHUNK = 32
STEPS = CHUNK // G


def _fused_kernel(cls_ref, loc_ref, gtb_ref, gtbt_ref, gtl_ref, db_ref,
                  out_ref, acc_ref, negs_ref, ks_ref):
    step = pl.program_id(0)
    stat_iota = jax.lax.broadcasted_iota(jnp.int32, (1, 128), 1)

    pneg = negs_ref[...]
    k_f = ks_ref[:, 0:1]
    k_i = k_f.astype(jnp.int32)
    nb = jax.lax.bitcast_convert_type(pneg, jnp.int32)

    t_bits = jnp.zeros((G, 1), jnp.int32)
    for i in range(31):
        cand = t_bits | (1 << (30 - i))
        cnt = jnp.sum((nb >= cand).astype(jnp.int32), axis=1, keepdims=True)
        t_bits = jnp.where(cnt >= k_i, cand, t_bits)
    t = jax.lax.bitcast_convert_type(t_bits, jnp.float32)

    gt_mask = pneg > t
    cnt_gt = jnp.sum(gt_mask.astype(jnp.float32), axis=1, keepdims=True)
    sum_gt = jnp.sum(jnp.where(gt_mask, pneg, 0.0), axis=1, keepdims=True)
    hard = jnp.sum(jnp.where(k_i > 0, sum_gt + (k_f - cnt_gt) * t, 0.0))
    hard = jnp.where(step > 0, hard, 0.0)

    cls = cls_ref[...]
    locp = loc_ref[...]
    gtb = gtb_ref[...]
    gtbt = gtbt_ref[...]
    gtl = gtl_ref[...]

    db = db_ref[...]
    cxd = db[0:1, :]
    cyd = db[1:2, :]
    wd = db[2:3, :]
    hd = db[3:4, :]
    x1d = (cxd - wd * 0.5)[:, None, :]
    y1d = (cyd - hd * 0.5)[:, None, :]
    x2d = (cxd + wd * 0.5)[:, None, :]
    y2d = (cyd + hd * 0.5)[:, None, :]
    area_d = (x2d - x1d) * (y2d - y1d)

    x1g = gtb[:, :, 0:1]
    y1g = gtb[:, :, 1:2]
    x2g = gtb[:, :, 2:3]
    y2g = gtb[:, :, 3:4]
    area_g = (x2g - x1g) * (y2g - y1g)

    iw = jnp.maximum(jnp.minimum(x2g, x2d) - jnp.maximum(x1g, x1d), 0.0)
    ih = jnp.maximum(jnp.minimum(y2g, y2d) - jnp.maximum(y1g, y1d), 0.0)
    inter = iw * ih
    ov = inter / (area_g + area_d - inter)

    obj_iota = jax.lax.broadcasted_iota(jnp.int32, (1, NOBJ, 1), 1)
    lane_iota3 = jax.lax.broadcasted_iota(jnp.int32, (1, 1, D), 2)

    bto_k = jnp.max(ov, axis=1, keepdims=True)
    bti = jnp.min(jnp.where(ov == bto_k, obj_iota, NOBJ), axis=1)
    bto = bto_k[:, 0, :]

    row_max = jnp.max(ov, axis=2, keepdims=True)
    bdi = jnp.min(jnp.where(ov == row_max, lane_iota3, D), axis=2,
                  keepdims=True)

    hit = lane_iota3 == bdi
    override = jnp.max(jnp.where(hit, obj_iota, -1), axis=1)
    forced = override >= 0
    bti = jnp.where(forced, override, bti)
    bto = jnp.where(forced, 1.0, bto)

    sel = (bti[:, None, :] == obj_iota).astype(jnp.float32)
    labels = jax.lax.dot_general(
        gtl, sel, (((2,), (1,)), ((0,), (0,))),
        preferred_element_type=jnp.float32)[:, 0, :]
    g = jax.lax.dot_general(
        gtbt, sel, (((2,), (1,)), ((0,), (0,))),
        preferred_element_type=jnp.float32)

    conf = jnp.where(bto < THRESHOLD, 0, labels.astype(jnp.int32))
    pos = conf > 0
    posf = pos.astype(jnp.float32)

    ri = jax.lax.broadcasted_iota(jnp.int32, (4, 4), 0)
    ci = jax.lax.broadcasted_iota(jnp.int32, (4, 4), 1)
    amat = jnp.where((ri & 1) == (ci & 1),
                     jnp.where(ri < 2, 0.5,
                               jnp.where(ci >= 2, 1.0, -1.0)),
                     0.0).astype(jnp.float32)
    amat3 = jnp.broadcast_to(amat[None], (G, 4, 4))
    u = jax.lax.dot_general(
        amat3, g, (((2,), (1,)), ((0,), (0,))),
        preferred_element_type=jnp.float32)

    dbwh = jnp.concatenate([db[2:4, :], db[2:4, :]], axis=0)
    row_iota = jax.lax.broadcasted_iota(jnp.int32, (1, 4, 1), 1)
    tmat = jnp.where(row_iota < 2,
                     (u - db[None]) / (dbwh[None] / 10.0),
                     jnp.log(u / dbwh[None]) * 5.0)

    d4 = locp - tmat
    ad4 = jnp.abs(d4)
    sl1v = jnp.where(ad4 < 1.0, 0.5 * ad4 * ad4, ad4 - 0.5)
    sl1_sum = jnp.sum(sl1v * posf[:, None, :])

    m = jnp.max(cls, axis=1, keepdims=True)
    e = jnp.exp(cls - m)
    ones_row = jnp.ones((G, 1, C), jnp.float32)
    s = jax.lax.dot_general(
        ones_row, e, (((2,), (1,)), ((0,), (0,))),
        preferred_element_type=jnp.float32)
    lse = m[:, 0, :] + jnp.log(s[:, 0, :])
    cls_iota = jax.lax.broadcasted_iota(jnp.int32, (1, C, 1), 1)
    gathered = jnp.sum(jnp.where(conf[:, None, :] == cls_iota, cls, 0.0),
                       axis=1)
    ce = lse - gathered

    npos = jnp.sum(posf, axis=1, keepdims=True)
    pos_loss = jnp.sum(jnp.where(pos, ce, 0.0))
    neg = jnp.where(pos, 0.0, ce)

    negs_ref[...] = neg
    ks_ref[...] = jnp.broadcast_to(NEG_POS * npos, (G, 128))

    n_pos_step = jnp.sum(npos)

    live = (step < STEPS).astype(jnp.float32)
    row = (jnp.where(stat_iota == 0, n_pos_step, 0.0)
           + jnp.where(stat_iota == 1, pos_loss, 0.0)
           + jnp.where(stat_iota == 2, sl1_sum, 0.0)) * live
    hard_row = jnp.where(stat_iota == 1, hard, 0.0)

    prev = jnp.where(step == 0, 0.0, acc_ref[...])
    accv = prev + row + hard_row
    acc_ref[...] = accv
    out_ref[...] = accv


def _combine_kernel(acc_a_ref, acc_b_ref, out_ref):
    stat_iota = jax.lax.broadcasted_iota(jnp.int32, (1, 128), 1)
    accv = acc_a_ref[...] + acc_b_ref[...]
    n_pos_total = jnp.sum(jnp.where(stat_iota == 0, accv, 0.0))
    conf_sum = jnp.sum(jnp.where(stat_iota == 1, accv, 0.0))
    sl1_total = jnp.sum(jnp.where(stat_iota == 2, accv, 0.0))
    loss = (ALPHA * sl1_total / (n_pos_total * 4.0)
            + conf_sum / n_pos_total)
    out_ref[...] = loss * jnp.ones((1, 128), jnp.float32)


@functools.partial(jax.jit, static_argnames=("interpret",))
def kernel(loc_pred, cls_pred, gt_boxes, gt_labels, default_boxes,
           interpret=False):
    db_t = jnp.transpose(default_boxes)

    def clamp(b):
        return jnp.minimum(b, STEPS - 1)

    def chunk_acc(lo):
        hi = lo + CHUNK
        cls_t = jnp.transpose(cls_pred[lo:hi], (0, 2, 1))
        loc_t = jnp.transpose(loc_pred[lo:hi], (0, 2, 1))
        gtb = gt_boxes[lo:hi]
        gtb_t = jnp.transpose(gtb, (0, 2, 1))
        gtl3 = gt_labels[lo:hi].astype(jnp.float32).reshape(CHUNK, 1, NOBJ)
        return pl.pallas_call(
            _fused_kernel,
            grid=(STEPS + 1,),
            in_specs=[
                pl.BlockSpec((G, C, D), lambda b: (clamp(b), 0, 0)),
                pl.BlockSpec((G, 4, D), lambda b: (clamp(b), 0, 0)),
                pl.BlockSpec((G, NOBJ, 4), lambda b: (clamp(b), 0, 0)),
                pl.BlockSpec((G, 4, NOBJ), lambda b: (clamp(b), 0, 0)),
                pl.BlockSpec((G, 1, NOBJ), lambda b: (clamp(b), 0, 0)),
                pl.BlockSpec((4, D), lambda b: (0, 0)),
            ],
            out_specs=pl.BlockSpec((1, 128), lambda b: (0, 0)),
            out_shape=jax.ShapeDtypeStruct((1, 128), jnp.float32),
            scratch_shapes=[
                pltpu.VMEM((1, 128), jnp.float32),
                pltpu.VMEM((G, D), jnp.float32),
                pltpu.VMEM((G, 128), jnp.float32),
            ],
            compiler_params=pltpu.CompilerParams(
                dimension_semantics=("arbitrary",)),
            interpret=interpret,
        )(cls_t, loc_t, gtb, gtb_t, gtl3, db_t)

    acc_a = chunk_acc(0)
    acc_b = chunk_acc(CHUNK)

    out = pl.pallas_call(
        _combine_kernel,
        interpret=interpret,
        out_shape=jax.ShapeDtypeStruct((1, 128), jnp.float32),
    )(acc_a, acc_b)

    return out[0, 0]


# speedup vs baseline: 1.1500x; 1.1500x over previous
import functools

import jax
import jax.numpy as jnp
from jax.experimental import pallas as pl
from jax.experimental.pallas import tpu as pltpu

B = 64
D = 8732
C = 21
NOBJ = 8
THRESHOLD = 0.5
NEG_POS = 3
ALPHA = 1.0

G = 8
STEPS = B // G


def _fused_kernel(cls_ref, loc_ref, gtb_ref, gtbt_ref, gtl_ref, db_ref,
                  out_ref, acc_ref, negs_ref, ks_ref):
    step = pl.program_id(0)
    stat_iota = jax.lax.broadcasted_iota(jnp.int32, (1, 128), 1)

    pneg = negs_ref[...]
    k_f = ks_ref[:, 0:1]
    k_i = k_f.astype(jnp.int32)
    nb = jax.lax.bitcast_convert_type(pneg, jnp.int32)

    t_bits = jnp.zeros((G, 1), jnp.int32)
    for i in range(31):
        cand = t_bits | (1 << (30 - i))
        cnt = jnp.sum((nb >= cand).astype(jnp.int32), axis=1, keepdims=True)
        t_bits = jnp.where(cnt >= k_i, cand, t_bits)
    t = jax.lax.bitcast_convert_type(t_bits, jnp.float32)

    gt_mask = pneg > t
    cnt_gt = jnp.sum(gt_mask.astype(jnp.float32), axis=1, keepdims=True)
    sum_gt = jnp.sum(jnp.where(gt_mask, pneg, 0.0), axis=1, keepdims=True)
    hard = jnp.sum(jnp.where(k_i > 0, sum_gt + (k_f - cnt_gt) * t, 0.0))
    hard = jnp.where(step > 0, hard, 0.0)

    cls = cls_ref[...]
    locp = loc_ref[...]
    gtb = gtb_ref[...]
    gtbt = gtbt_ref[...]
    gtl = gtl_ref[...]

    db = db_ref[...]
    cxd = db[0:1, :]
    cyd = db[1:2, :]
    wd = db[2:3, :]
    hd = db[3:4, :]
    x1d = (cxd - wd * 0.5)[:, None, :]
    y1d = (cyd - hd * 0.5)[:, None, :]
    x2d = (cxd + wd * 0.5)[:, None, :]
    y2d = (cyd + hd * 0.5)[:, None, :]
    area_d = (x2d - x1d) * (y2d - y1d)

    x1g = gtb[:, :, 0:1]
    y1g = gtb[:, :, 1:2]
    x2g = gtb[:, :, 2:3]
    y2g = gtb[:, :, 3:4]
    area_g = (x2g - x1g) * (y2g - y1g)

    iw = jnp.maximum(jnp.minimum(x2g, x2d) - jnp.maximum(x1g, x1d), 0.0)
    ih = jnp.maximum(jnp.minimum(y2g, y2d) - jnp.maximum(y1g, y1d), 0.0)
    inter = iw * ih
    ov = inter / (area_g + area_d - inter)

    obj_iota = jax.lax.broadcasted_iota(jnp.int32, (1, NOBJ, 1), 1)
    lane_iota3 = jax.lax.broadcasted_iota(jnp.int32, (1, 1, D), 2)

    bto_k = jnp.max(ov, axis=1, keepdims=True)
    bti = jnp.min(jnp.where(ov == bto_k, obj_iota, NOBJ), axis=1)
    bto = bto_k[:, 0, :]

    row_max = jnp.max(ov, axis=2, keepdims=True)
    bdi = jnp.min(jnp.where(ov == row_max, lane_iota3, D), axis=2,
                  keepdims=True)

    hit = lane_iota3 == bdi
    override = jnp.max(jnp.where(hit, obj_iota, -1), axis=1)
    forced = override >= 0
    bti = jnp.where(forced, override, bti)
    bto = jnp.where(forced, 1.0, bto)

    sel = (bti[:, None, :] == obj_iota).astype(jnp.float32)
    labels = jax.lax.dot_general(
        gtl, sel, (((2,), (1,)), ((0,), (0,))),
        preferred_element_type=jnp.float32)[:, 0, :]
    g = jax.lax.dot_general(
        gtbt, sel, (((2,), (1,)), ((0,), (0,))),
        preferred_element_type=jnp.float32)

    conf = jnp.where(bto < THRESHOLD, 0, labels.astype(jnp.int32))
    pos = conf > 0
    posf = pos.astype(jnp.float32)

    ri = jax.lax.broadcasted_iota(jnp.int32, (4, 4), 0)
    ci = jax.lax.broadcasted_iota(jnp.int32, (4, 4), 1)
    amat = jnp.where((ri & 1) == (ci & 1),
                     jnp.where(ri < 2, 0.5,
                               jnp.where(ci >= 2, 1.0, -1.0)),
                     0.0).astype(jnp.float32)
    amat3 = jnp.broadcast_to(amat[None], (G, 4, 4))
    u = jax.lax.dot_general(
        amat3, g, (((2,), (1,)), ((0,), (0,))),
        preferred_element_type=jnp.float32)

    dbwh = jnp.concatenate([db[2:4, :], db[2:4, :]], axis=0)
    row_iota = jax.lax.broadcasted_iota(jnp.int32, (1, 4, 1), 1)
    tmat = jnp.where(row_iota < 2,
                     (u - db[None]) / (dbwh[None] / 10.0),
                     jnp.log(u / dbwh[None]) * 5.0)

    d4 = locp - tmat
    ad4 = jnp.abs(d4)
    sl1v = jnp.where(ad4 < 1.0, 0.5 * ad4 * ad4, ad4 - 0.5)
    sl1_sum = jnp.sum(sl1v * posf[:, None, :])

    m = jnp.max(cls, axis=1, keepdims=True)
    e = jnp.exp(cls - m)
    ones_row = jnp.ones((G, 1, C), jnp.float32)
    s = jax.lax.dot_general(
        ones_row, e, (((2,), (1,)), ((0,), (0,))),
        preferred_element_type=jnp.float32)
    lse = m[:, 0, :] + jnp.log(s[:, 0, :])
    cls_iota = jax.lax.broadcasted_iota(jnp.int32, (1, C, 1), 1)
    gathered = jnp.sum(jnp.where(conf[:, None, :] == cls_iota, cls, 0.0),
                       axis=1)
    ce = lse - gathered

    npos = jnp.sum(posf, axis=1, keepdims=True)
    pos_loss = jnp.sum(jnp.where(pos, ce, 0.0))
    neg = jnp.where(pos, 0.0, ce)

    negs_ref[...] = neg
    ks_ref[...] = jnp.broadcast_to(NEG_POS * npos, (G, 128))

    n_pos_step = jnp.sum(npos)

    live = (step < STEPS).astype(jnp.float32)
    row = (jnp.where(stat_iota == 0, n_pos_step, 0.0)
           + jnp.where(stat_iota == 1, pos_loss, 0.0)
           + jnp.where(stat_iota == 2, sl1_sum, 0.0)) * live
    hard_row = jnp.where(stat_iota == 1, hard, 0.0)

    prev = jnp.where(step == 0, 0.0, acc_ref[...])
    accv = prev + row + hard_row
    acc_ref[...] = accv

    n_pos_total = jnp.sum(jnp.where(stat_iota == 0, accv, 0.0))
    conf_sum = jnp.sum(jnp.where(stat_iota == 1, accv, 0.0))
    sl1_total = jnp.sum(jnp.where(stat_iota == 2, accv, 0.0))
    loss = (ALPHA * sl1_total / (n_pos_total * 4.0)
            + conf_sum / n_pos_total)
    out_ref[...] = loss * jnp.ones((1, 128), jnp.float32)


@functools.partial(jax.jit, static_argnames=("interpret",))
def kernel(loc_pred, cls_pred, gt_boxes, gt_labels, default_boxes,
           interpret=False):
    cls_t = jnp.transpose(cls_pred, (0, 2, 1))
    loc_t = jnp.transpose(loc_pred, (0, 2, 1))
    db_t = jnp.transpose(default_boxes)
    gtb_t = jnp.transpose(gt_boxes, (0, 2, 1))
    gtl3 = gt_labels.astype(jnp.float32).reshape(B, 1, NOBJ)

    def clamp(b):
        return jnp.minimum(b, STEPS - 1)

    out = pl.pallas_call(
        _fused_kernel,
        grid=(STEPS + 1,),
        in_specs=[
            pl.BlockSpec((G, C, D), lambda b: (clamp(b), 0, 0)),
            pl.BlockSpec((G, 4, D), lambda b: (clamp(b), 0, 0)),
            pl.BlockSpec((G, NOBJ, 4), lambda b: (clamp(b), 0, 0)),
            pl.BlockSpec((G, 4, NOBJ), lambda b: (clamp(b), 0, 0)),
            pl.BlockSpec((G, 1, NOBJ), lambda b: (clamp(b), 0, 0)),
            pl.BlockSpec((4, D), lambda b: (0, 0)),
        ],
        out_specs=pl.BlockSpec((1, 128), lambda b: (0, 0)),
        out_shape=jax.ShapeDtypeStruct((1, 128), jnp.float32),
        scratch_shapes=[
            pltpu.VMEM((1, 128), jnp.float32),
            pltpu.VMEM((G, D), jnp.float32),
            pltpu.VMEM((G, 128), jnp.float32),
        ],
        compiler_params=pltpu.CompilerParams(
            dimension_semantics=("arbitrary",)),
        interpret=interpret,
    )(cls_t, loc_t, gt_boxes, gtb_t, gtl3, db_t)

    return out[0, 0]


# speedup vs baseline: 1.1814x; 1.0274x over previous
import functools

import jax
import jax.numpy as jnp
from jax.experimental import pallas as pl
from jax.experimental.pallas import tpu as pltpu

B = 64
D = 8732
C = 21
NOBJ = 8
THRESHOLD = 0.5
NEG_POS = 3
ALPHA = 1.0

G = 16
STEPS = B // G


def _fused_kernel(cls_ref, loc_ref, gtb_ref, gtbt_ref, gtl_ref, db_ref,
                  out_ref, acc_ref, negs_ref, ks_ref):
    step = pl.program_id(0)
    stat_iota = jax.lax.broadcasted_iota(jnp.int32, (1, 128), 1)

    pneg = negs_ref[...]
    k_f = ks_ref[:, 0:1]
    k_i = k_f.astype(jnp.int32)
    nb = jax.lax.bitcast_convert_type(pneg, jnp.int32)

    t_bits = jnp.zeros((G, 1), jnp.int32)
    for i in range(31):
        cand = t_bits | (1 << (30 - i))
        cnt = jnp.sum((nb >= cand).astype(jnp.int32), axis=1, keepdims=True)
        t_bits = jnp.where(cnt >= k_i, cand, t_bits)
    t = jax.lax.bitcast_convert_type(t_bits, jnp.float32)

    gt_mask = pneg > t
    cnt_gt = jnp.sum(gt_mask.astype(jnp.float32), axis=1, keepdims=True)
    sum_gt = jnp.sum(jnp.where(gt_mask, pneg, 0.0), axis=1, keepdims=True)
    hard = jnp.sum(jnp.where(k_i > 0, sum_gt + (k_f - cnt_gt) * t, 0.0))
    hard = jnp.where(step > 0, hard, 0.0)

    cls = cls_ref[...]
    locp = loc_ref[...]
    gtb = gtb_ref[...]
    gtbt = gtbt_ref[...]
    gtl = gtl_ref[...]

    db = db_ref[...]
    cxd = db[0:1, :]
    cyd = db[1:2, :]
    wd = db[2:3, :]
    hd = db[3:4, :]
    x1d = (cxd - wd * 0.5)[:, None, :]
    y1d = (cyd - hd * 0.5)[:, None, :]
    x2d = (cxd + wd * 0.5)[:, None, :]
    y2d = (cyd + hd * 0.5)[:, None, :]
    area_d = (x2d - x1d) * (y2d - y1d)

    x1g = gtb[:, :, 0:1]
    y1g = gtb[:, :, 1:2]
    x2g = gtb[:, :, 2:3]
    y2g = gtb[:, :, 3:4]
    area_g = (x2g - x1g) * (y2g - y1g)

    iw = jnp.maximum(jnp.minimum(x2g, x2d) - jnp.maximum(x1g, x1d), 0.0)
    ih = jnp.maximum(jnp.minimum(y2g, y2d) - jnp.maximum(y1g, y1d), 0.0)
    inter = iw * ih
    ov = inter / (area_g + area_d - inter)

    obj_iota = jax.lax.broadcasted_iota(jnp.int32, (1, NOBJ, 1), 1)
    lane_iota3 = jax.lax.broadcasted_iota(jnp.int32, (1, 1, D), 2)

    bto_k = jnp.max(ov, axis=1, keepdims=True)
    bti = jnp.min(jnp.where(ov == bto_k, obj_iota, NOBJ), axis=1)
    bto = bto_k[:, 0, :]

    row_max = jnp.max(ov, axis=2, keepdims=True)
    bdi = jnp.min(jnp.where(ov == row_max, lane_iota3, D), axis=2,
                  keepdims=True)

    hit = lane_iota3 == bdi
    override = jnp.max(jnp.where(hit, obj_iota, -1), axis=1)
    forced = override >= 0
    bti = jnp.where(forced, override, bti)
    bto = jnp.where(forced, 1.0, bto)

    sel = (bti[:, None, :] == obj_iota).astype(jnp.float32)
    labels = jax.lax.dot_general(
        gtl, sel, (((2,), (1,)), ((0,), (0,))),
        preferred_element_type=jnp.float32)[:, 0, :]
    g = jax.lax.dot_general(
        gtbt, sel, (((2,), (1,)), ((0,), (0,))),
        preferred_element_type=jnp.float32)

    conf = jnp.where(bto < THRESHOLD, 0, labels.astype(jnp.int32))
    pos = conf > 0
    posf = pos.astype(jnp.float32)

    ri = jax.lax.broadcasted_iota(jnp.int32, (4, 4), 0)
    ci = jax.lax.broadcasted_iota(jnp.int32, (4, 4), 1)
    amat = jnp.where((ri & 1) == (ci & 1),
                     jnp.where(ri < 2, 0.5,
                               jnp.where(ci >= 2, 1.0, -1.0)),
                     0.0).astype(jnp.float32)
    amat3 = jnp.broadcast_to(amat[None], (G, 4, 4))
    u = jax.lax.dot_general(
        amat3, g, (((2,), (1,)), ((0,), (0,))),
        preferred_element_type=jnp.float32)

    dbwh = jnp.concatenate([db[2:4, :], db[2:4, :]], axis=0)
    row_iota = jax.lax.broadcasted_iota(jnp.int32, (1, 4, 1), 1)
    tmat = jnp.where(row_iota < 2,
                     (u - db[None]) / (dbwh[None] / 10.0),
                     jnp.log(u / dbwh[None]) * 5.0)

    d4 = locp - tmat
    ad4 = jnp.abs(d4)
    sl1v = jnp.where(ad4 < 1.0, 0.5 * ad4 * ad4, ad4 - 0.5)
    sl1_sum = jnp.sum(sl1v * posf[:, None, :])

    m = jnp.max(cls, axis=1, keepdims=True)
    e = jnp.exp(cls - m)
    ones_row = jnp.ones((G, 1, C), jnp.float32)
    s = jax.lax.dot_general(
        ones_row, e, (((2,), (1,)), ((0,), (0,))),
        preferred_element_type=jnp.float32)
    lse = m[:, 0, :] + jnp.log(s[:, 0, :])
    cls_iota = jax.lax.broadcasted_iota(jnp.int32, (1, C, 1), 1)
    gathered = jnp.sum(jnp.where(conf[:, None, :] == cls_iota, cls, 0.0),
                       axis=1)
    ce = lse - gathered

    npos = jnp.sum(posf, axis=1, keepdims=True)
    pos_loss = jnp.sum(jnp.where(pos, ce, 0.0))
    neg = jnp.where(pos, 0.0, ce)

    negs_ref[...] = neg
    ks_ref[...] = jnp.broadcast_to(NEG_POS * npos, (G, 128))

    n_pos_step = jnp.sum(npos)

    live = (step < STEPS).astype(jnp.float32)
    row = (jnp.where(stat_iota == 0, n_pos_step, 0.0)
           + jnp.where(stat_iota == 1, pos_loss, 0.0)
           + jnp.where(stat_iota == 2, sl1_sum, 0.0)) * live
    hard_row = jnp.where(stat_iota == 1, hard, 0.0)

    prev = jnp.where(step == 0, 0.0, acc_ref[...])
    accv = prev + row + hard_row
    acc_ref[...] = accv

    n_pos_total = jnp.sum(jnp.where(stat_iota == 0, accv, 0.0))
    conf_sum = jnp.sum(jnp.where(stat_iota == 1, accv, 0.0))
    sl1_total = jnp.sum(jnp.where(stat_iota == 2, accv, 0.0))
    loss = (ALPHA * sl1_total / (n_pos_total * 4.0)
            + conf_sum / n_pos_total)
    out_ref[...] = loss * jnp.ones((1, 128), jnp.float32)


@functools.partial(jax.jit, static_argnames=("interpret",))
def kernel(loc_pred, cls_pred, gt_boxes, gt_labels, default_boxes,
           interpret=False):
    cls_t = jnp.transpose(cls_pred, (0, 2, 1))
    loc_t = jnp.transpose(loc_pred, (0, 2, 1))
    db_t = jnp.transpose(default_boxes)
    gtb_t = jnp.transpose(gt_boxes, (0, 2, 1))
    gtl3 = gt_labels.astype(jnp.float32).reshape(B, 1, NOBJ)

    def clamp(b):
        return jnp.minimum(b, STEPS - 1)

    out = pl.pallas_call(
        _fused_kernel,
        grid=(STEPS + 1,),
        in_specs=[
            pl.BlockSpec((G, C, D), lambda b: (clamp(b), 0, 0)),
            pl.BlockSpec((G, 4, D), lambda b: (clamp(b), 0, 0)),
            pl.BlockSpec((G, NOBJ, 4), lambda b: (clamp(b), 0, 0)),
            pl.BlockSpec((G, 4, NOBJ), lambda b: (clamp(b), 0, 0)),
            pl.BlockSpec((G, 1, NOBJ), lambda b: (clamp(b), 0, 0)),
            pl.BlockSpec((4, D), lambda b: (0, 0)),
        ],
        out_specs=pl.BlockSpec((1, 128), lambda b: (0, 0)),
        out_shape=jax.ShapeDtypeStruct((1, 128), jnp.float32),
        scratch_shapes=[
            pltpu.VMEM((1, 128), jnp.float32),
            pltpu.VMEM((G, D), jnp.float32),
            pltpu.VMEM((G, 128), jnp.float32),
        ],
        compiler_params=pltpu.CompilerParams(
            dimension_semantics=("arbitrary",)),
        interpret=interpret,
    )(cls_t, loc_t, gt_boxes, gtb_t, gtl3, db_t)

    return out[0, 0]
